# baseline (speedup 1.0000x reference)
import functools

import numpy as np
import jax
import jax.numpy as jnp
from jax import lax
from jax.experimental import pallas as pl
from jax.experimental.pallas import tpu as pltpu
from jax.experimental.pallas import tpu_sc as plsc

_STRIDES = [8, 16, 32, 64, 128]
_LIMITS = [[-1, 64], [64, 128], [128, 256], [256, 512], [512, 999999]]
_LEVEL_HW = [(64, 64), (32, 32), (16, 16), (8, 8), (4, 4)]
_B, _M = 8, 64
_NLOC = sum(h * w for h, w in _LEVEL_HW)
_NLOCP = 5504
_NQ = 4
_QL = _NLOCP // _NQ
_NCHQ = 88
_LW = _NCHQ * 16
_NCHT = _NQ * _NCHQ
_BIG = np.float32(99999999.0)


def _pad_quarters(a, padval):
    a = a.reshape(_NQ, _QL)
    pad = np.full((_NQ, _LW - _QL), padval, np.float32)
    return np.concatenate([a, pad], axis=1).reshape(-1)


def _build_loc_tables():
    xs, ys, lo, hi, rad = [], [], [], [], []
    for (h, w), s, (llo, lhi) in zip(_LEVEL_HW, _STRIDES, _LIMITS):
        ix = np.arange(w, dtype=np.float32) * s + s // 2
        iy = np.arange(h, dtype=np.float32) * s + s // 2
        xs.append(np.tile(ix, h))
        ys.append(np.repeat(iy, w))
        lo.append(np.full(h * w, llo, np.float32))
        hi.append(np.full(h * w, lhi, np.float32))
        rad.append(np.full(h * w, s * 1.5, np.float32))
    pad = _NLOCP - _NLOC
    out = []
    for arrs, padval in zip((xs, ys, lo, hi, rad), (0.0, 0.0, 1e9, -1e9, 0.0)):
        a = np.concatenate(arrs)
        a = np.concatenate([a, np.full(pad, padval, np.float32)])
        out.append(_pad_quarters(a, padval))
    return out


_XS, _YS, _LO, _HI, _RAD = _build_loc_tables()


def _build_chunk_tables():
    xs2 = _XS.reshape(-1, 16)
    ys2 = _YS.reshape(-1, 16)
    xc = (xs2.min(axis=1) + xs2.max(axis=1)) * 0.5
    hsx = (xs2.max(axis=1) - xs2.min(axis=1)) * 0.5 + 0.5
    yc = (ys2.min(axis=1) + ys2.max(axis=1)) * 0.5
    hsy = (ys2.max(axis=1) - ys2.min(axis=1)) * 0.5 + 0.5
    rad = _RAD.reshape(-1, 16)[:, 0]
    lo = _LO.reshape(-1, 16)[:, 0]
    hi = _HI.reshape(-1, 16)[:, 0]
    f32 = lambda a: a.astype(np.float32)
    return tuple(map(f32, (xc, yc, hsx, hsy, rad, lo, hi)))


_XCC, _YCC, _HSXC, _HSYC, _RADC, _LOC, _HIC = _build_chunk_tables()
_POW32 = (np.uint32(1) << np.arange(32, dtype=np.uint32)).astype(np.uint32)


def _sc_body(xs_ref, ys_ref, lo_ref, hi_ref, rad_ref, gtb_ref, cls_ref,
             mw_ref,
             cls_out, cnt_out, reg_out,
             x_v, y_v, lo_v, hi_v, rad_v, gtb_v, clsr_v, boxf_v, clsb_v,
             mw_v, ocls_v, ocnt_v, orl_v, ort_v, orr_v, orb_v):
    cid = lax.axis_index("c")
    sid = lax.axis_index("s")
    wid = sid * 2 + cid
    b = wid // _NQ
    q = wid % _NQ
    base = q * _LW

    pltpu.sync_copy(xs_ref.at[pl.ds(base, _LW)], x_v)
    pltpu.sync_copy(ys_ref.at[pl.ds(base, _LW)], y_v)
    pltpu.sync_copy(lo_ref.at[pl.ds(base, _LW)], lo_v)
    pltpu.sync_copy(hi_ref.at[pl.ds(base, _LW)], hi_v)
    pltpu.sync_copy(rad_ref.at[pl.ds(base, _LW)], rad_v)
    pltpu.sync_copy(gtb_ref.at[pl.ds(b * 4 * _M, 4 * _M)], gtb_v)
    pltpu.sync_copy(cls_ref.at[pl.ds(b * _M, _M)], clsr_v)
    pltpu.sync_copy(mw_ref.at[pl.ds((b * _NQ + q) * 2 * _NCHQ, 2 * _NCHQ)],
                    mw_v.at[pl.ds(0, 2 * _NCHQ)])

    for gi in range(_M // 4):
        v16 = gtb_v[pl.ds(gi * 16, 16)]
        for j in range(4):
            m = gi * 4 + j
            x0 = v16[4 * j]
            y1 = v16[4 * j + 1]
            x2 = v16[4 * j + 2]
            y3 = v16[4 * j + 3]
            boxf_v[pl.ds(m * 16, 16)] = jnp.full((16,), x0, jnp.float32)
            boxf_v[pl.ds(m * 16 + _M * 16, 16)] = jnp.full((16,), y1,
                                                           jnp.float32)
            boxf_v[pl.ds(m * 16 + 2 * _M * 16, 16)] = jnp.full((16,), x2,
                                                               jnp.float32)
            boxf_v[pl.ds(m * 16 + 3 * _M * 16, 16)] = jnp.full((16,), y3,
                                                               jnp.float32)
            boxf_v[pl.ds(m * 16 + 4 * _M * 16, 16)] = jnp.full(
                (16,), (x0 + x2) * 0.5, jnp.float32)
            boxf_v[pl.ds(m * 16 + 5 * _M * 16, 16)] = jnp.full(
                (16,), (y1 + y3) * 0.5, jnp.float32)
    for gi in range(_M // 16):
        c16 = clsr_v[pl.ds(gi * 16, 16)]
        for j in range(16):
            m = gi * 16 + j
            clsb_v[pl.ds(m * 16, 16)] = jnp.full((16,), c16[j], jnp.int32)

    big = jnp.full((16,), _BIG, jnp.float32)

    def chunk(i, carry):
        s16 = pl.ds(i * 16, 16)
        xv = x_v[s16]
        yv = y_v[s16]
        lov = lo_v[s16]
        hiv = hi_v[s16]
        radv = rad_v[s16]
        w16 = mw_v[pl.ds(i * 2, 16)]
        m_a = w16[0]
        m_b = w16[1]

        zero = jnp.zeros((16,), jnp.float32)
        state = (jnp.full((16,), 2e8, jnp.float32), zero, zero, zero, zero,
                 jnp.zeros((16,), jnp.int32))

        def one_box(mb, st):
            best, sl, stt, sr, sb, scl = st
            x0 = boxf_v[pl.ds(mb, 16)]
            y1 = boxf_v[pl.ds(mb + _M * 16, 16)]
            x2 = boxf_v[pl.ds(mb + 2 * _M * 16, 16)]
            y3 = boxf_v[pl.ds(mb + 3 * _M * 16, 16)]
            cx = boxf_v[pl.ds(mb + 4 * _M * 16, 16)]
            cy = boxf_v[pl.ds(mb + 5 * _M * 16, 16)]
            cl = clsb_v[pl.ds(mb, 16)]
            l = xv - x0
            t = yv - y1
            r = x2 - xv
            bb = y3 - yv
            area = (l + r) * (t + bb)
            mn = jnp.minimum(jnp.minimum(l, t), jnp.minimum(r, bb))
            mx = jnp.maximum(jnp.maximum(l, t), jnp.maximum(r, bb))
            dm = jnp.maximum(jnp.abs(xv - cx), jnp.abs(yv - cy))
            mask = (mn > 0.0) & (mx > lov) & (mx <= hiv) & (dm < radv)
            am = jnp.where(mask, area, big)
            take = am < best
            best = jnp.where(take, am, best)
            sl = jnp.where(take, l, sl)
            stt = jnp.where(take, t, stt)
            sr = jnp.where(take, r, sr)
            sb = jnp.where(take, bb, sb)
            scl = jnp.where(take, cl, scl)
            return best, sl, stt, sr, sb, scl

        def popcount(m0):
            x = m0 - (lax.shift_right_logical(m0, 1) & 0x55555555)
            x = (x & 0x33333333) + (lax.shift_right_logical(x, 2) & 0x33333333)
            x = (x + lax.shift_right_logical(x, 4)) & 0x0F0F0F0F
            return lax.shift_right_logical(x * 0x01010101, 24)

        def bit_loop(moffset16):
            def body(k, carry):
                m0 = carry[0]
                low = m0 & (-m0)
                fb = lax.bitcast_convert_type(
                    lax.convert_element_type(low, jnp.float32), jnp.int32)
                bi = ((fb >> 23) & 255) - 127
                st2 = one_box(bi * 16 + moffset16, carry[1:])
                return (m0 ^ low,) + st2
            return body

        res = lax.fori_loop(0, popcount(m_a), bit_loop(0), (m_a,) + state)
        res = lax.fori_loop(0, popcount(m_b), bit_loop(_M // 2 * 16),
                            (m_b,) + res[1:])
        best, sl, stt, sr, sb, scl = res[1:]
        anyp = best < big
        lrmin = jnp.minimum(sl, sr)
        lrmax = jnp.maximum(sl, sr)
        tbmin = jnp.minimum(stt, sb)
        tbmax = jnp.maximum(stt, sb)
        ratio = (lrmin * tbmin) / (lrmax * tbmax + 1e-10)
        s = jnp.maximum(jnp.maximum(ratio, 0.0), 1e-30)
        bits = lax.bitcast_convert_type(s, jnp.int32)
        yv0 = lax.bitcast_convert_type(jnp.int32(0x5F3759DF) - (bits >> 1),
                                       jnp.float32)
        for _ in range(3):
            yv0 = yv0 * (1.5 - 0.5 * s * yv0 * yv0)
        sq = s * yv0
        neg1 = jnp.full((16,), -1.0, jnp.float32)
        ocnt_v[s16] = jnp.where(anyp, sq, neg1)
        ocls_v[s16] = jnp.where(anyp, scl, jnp.zeros((16,), jnp.int32))
        orl_v[s16] = jnp.where(anyp, sl, neg1)
        ort_v[s16] = jnp.where(anyp, stt, neg1)
        orr_v[s16] = jnp.where(anyp, sr, neg1)
        orb_v[s16] = jnp.where(anyp, sb, neg1)
        return carry

    lax.fori_loop(0, _NCHQ, chunk, 0)

    obase = b * _NLOC + q * _QL
    rbase = b * 4 * _NLOC + q * _QL
    qtail = _NLOC - 3 * _QL

    def _emit(n):
        pltpu.sync_copy(ocls_v.at[pl.ds(0, n)], cls_out.at[pl.ds(obase, n)])
        pltpu.sync_copy(ocnt_v.at[pl.ds(0, n)], cnt_out.at[pl.ds(obase, n)])
        pltpu.sync_copy(orl_v.at[pl.ds(0, n)], reg_out.at[pl.ds(rbase, n)])
        pltpu.sync_copy(ort_v.at[pl.ds(0, n)],
                        reg_out.at[pl.ds(rbase + _NLOC, n)])
        pltpu.sync_copy(orr_v.at[pl.ds(0, n)],
                        reg_out.at[pl.ds(rbase + 2 * _NLOC, n)])
        pltpu.sync_copy(orb_v.at[pl.ds(0, n)],
                        reg_out.at[pl.ds(rbase + 3 * _NLOC, n)])

    @pl.when(q != _NQ - 1)
    def _():
        _emit(_QL)

    @pl.when(q == _NQ - 1)
    def _():
        _emit(qtail)


@functools.partial(
    pl.kernel,
    out_type=(
        jax.ShapeDtypeStruct((_B * _NLOC,), jnp.int32),
        jax.ShapeDtypeStruct((_B * _NLOC,), jnp.float32),
        jax.ShapeDtypeStruct((_B * 4 * _NLOC,), jnp.float32),
    ),
    mesh=plsc.VectorSubcoreMesh(core_axis_name="c", subcore_axis_name="s",
                                num_cores=2, num_subcores=16),
    scratch_types=(
        [pltpu.VMEM((_LW,), jnp.float32) for _ in range(5)]
        + [
            pltpu.VMEM((4 * _M,), jnp.float32),
            pltpu.VMEM((_M,), jnp.int32),
            pltpu.VMEM((6 * _M * 16,), jnp.float32),
            pltpu.VMEM((_M * 16,), jnp.int32),
            pltpu.VMEM((2 * _NCHQ + 16,), jnp.int32),
            pltpu.VMEM((_LW,), jnp.int32),
        ]
        + [pltpu.VMEM((_LW,), jnp.float32) for _ in range(5)]
    ),
)
def _gen_targets_sc(xs, ys, lo, hi, rad, gtb, cls_i, mw,
                    cls_out, cnt_out, reg_out, *scratch):
    _sc_body(xs, ys, lo, hi, rad, gtb, cls_i, mw,
             cls_out, cnt_out, reg_out, *scratch)


def kernel(cls_logits_0, cls_logits_1, cls_logits_2, cls_logits_3,
           cls_logits_4, cnt_logits_0, cnt_logits_1, cnt_logits_2,
           cnt_logits_3, cnt_logits_4, reg_preds_0, reg_preds_1, reg_preds_2,
           reg_preds_3, reg_preds_4, gt_boxes, classes):
    x0 = gt_boxes[..., 0]
    y1 = gt_boxes[..., 1]
    x2 = gt_boxes[..., 2]
    y3 = gt_boxes[..., 3]
    cx = (x0 + x2) / 2
    cy = (y1 + y3) / 2
    hw = (x2 - x0) * 0.5
    hh = (y3 - y1) * 0.5
    xcc = jnp.asarray(_XCC)[None, :, None]
    ycc = jnp.asarray(_YCC)[None, :, None]
    hsx = jnp.asarray(_HSXC)[None, :, None]
    hsy = jnp.asarray(_HSYC)[None, :, None]
    radc = jnp.asarray(_RADC)[None, :, None]
    loc = jnp.asarray(_LOC)[None, :, None]
    hic = jnp.asarray(_HIC)[None, :, None]
    cxb = cx[:, None, :]
    cyb = cy[:, None, :]
    hwb = hw[:, None, :]
    hhb = hh[:, None, :]
    act = ((jnp.abs(cxb - xcc) < jnp.minimum(radc, hwb) + hsx)
           & (jnp.abs(cyb - ycc) < jnp.minimum(radc, hhb) + hsy))
    mxwh = jnp.maximum(hwb, hhb)
    act = act & (mxwh > loc - radc - 1.0) & (mxwh <= hic + 1.0)
    pow32 = jnp.asarray(_POW32)
    m_a = jnp.sum(jnp.where(act[..., :32], pow32, jnp.uint32(0)), axis=-1,
                  dtype=jnp.uint32)
    m_b = jnp.sum(jnp.where(act[..., 32:], pow32, jnp.uint32(0)), axis=-1,
                  dtype=jnp.uint32)
    words = lax.bitcast_convert_type(jnp.stack([m_a, m_b], axis=-1),
                                     jnp.int32)
    mw = words.reshape(_B * _NCHT * 2)

    cls_flat, cnt_flat, reg_flat = _gen_targets_sc(
        jnp.asarray(_XS), jnp.asarray(_YS), jnp.asarray(_LO),
        jnp.asarray(_HI), jnp.asarray(_RAD),
        gt_boxes.reshape(_B * 4 * _M), classes.reshape(_B * _M), mw)

    cls_t = cls_flat.reshape(_B, _NLOC, 1)
    cnt_t = cnt_flat.reshape(_B, _NLOC, 1)
    reg_t = jnp.transpose(reg_flat.reshape(_B, 4, _NLOC), (0, 2, 1))
    return cls_t, cnt_t, reg_t


# speedup vs baseline: 1.0559x; 1.0559x over previous
import functools

import numpy as np
import jax
import jax.numpy as jnp
from jax import lax
from jax.experimental import pallas as pl
from jax.experimental.pallas import tpu as pltpu
from jax.experimental.pallas import tpu_sc as plsc

_STRIDES = [8, 16, 32, 64, 128]
_LIMITS = [[-1, 64], [64, 128], [128, 256], [256, 512], [512, 999999]]
_LEVEL_HW = [(64, 64), (32, 32), (16, 16), (8, 8), (4, 4)]
_B, _M = 8, 64
_NLOC = sum(h * w for h, w in _LEVEL_HW)
_NLOCP = 5504
_NQ = 4
_QL = _NLOCP // _NQ
_NCHQ = 88
_LW = _NCHQ * 16
_NCHT = _NQ * _NCHQ
_BIG = np.float32(99999999.0)


def _pad_quarters(a, padval):
    a = a.reshape(_NQ, _QL)
    pad = np.full((_NQ, _LW - _QL), padval, np.float32)
    return np.concatenate([a, pad], axis=1).reshape(-1)


def _build_loc_tables():
    xs, ys, lo, hi, rad = [], [], [], [], []
    for (h, w), s, (llo, lhi) in zip(_LEVEL_HW, _STRIDES, _LIMITS):
        ix = np.arange(w, dtype=np.float32) * s + s // 2
        iy = np.arange(h, dtype=np.float32) * s + s // 2
        xs.append(np.tile(ix, h))
        ys.append(np.repeat(iy, w))
        lo.append(np.full(h * w, llo, np.float32))
        hi.append(np.full(h * w, lhi, np.float32))
        rad.append(np.full(h * w, s * 1.5, np.float32))
    pad = _NLOCP - _NLOC
    out = []
    for arrs, padval in zip((xs, ys, lo, hi, rad), (0.0, 0.0, 1e9, -1e9, 0.0)):
        a = np.concatenate(arrs)
        a = np.concatenate([a, np.full(pad, padval, np.float32)])
        out.append(_pad_quarters(a, padval))
    return out


_XS, _YS, _LO, _HI, _RAD = _build_loc_tables()


def _build_chunk_tables():
    xs2 = _XS.reshape(-1, 16)
    ys2 = _YS.reshape(-1, 16)
    xc = (xs2.min(axis=1) + xs2.max(axis=1)) * 0.5
    hsx = (xs2.max(axis=1) - xs2.min(axis=1)) * 0.5 + 0.5
    yc = (ys2.min(axis=1) + ys2.max(axis=1)) * 0.5
    hsy = (ys2.max(axis=1) - ys2.min(axis=1)) * 0.5 + 0.5
    rad = _RAD.reshape(-1, 16)[:, 0]
    lo = _LO.reshape(-1, 16)[:, 0]
    hi = _HI.reshape(-1, 16)[:, 0]
    f32 = lambda a: a.astype(np.float32)
    return tuple(map(f32, (xc, yc, hsx, hsy, rad, lo, hi)))


_XCC, _YCC, _HSXC, _HSYC, _RADC, _LOC, _HIC = _build_chunk_tables()
_POW32 = (np.uint32(1) << np.arange(32, dtype=np.uint32)).astype(np.uint32)


def _sc_body(xs_ref, ys_ref, lo_ref, hi_ref, rad_ref, gtb_ref, cls_ref,
             mw_ref,
             cls_out, cnt_out, reg_out,
             x_v, y_v, lo_v, hi_v, rad_v, gtb_v, clsr_v, boxf_v, clsb_v,
             mw_v, ocls_v, ocnt_v, orl_v, ort_v, orr_v, orb_v):
    cid = lax.axis_index("c")
    sid = lax.axis_index("s")
    wid = sid * 2 + cid
    b = wid // _NQ
    q = wid % _NQ
    base = q * _LW

    pltpu.sync_copy(xs_ref.at[pl.ds(base, _LW)], x_v)
    pltpu.sync_copy(ys_ref.at[pl.ds(base, _LW)], y_v)
    pltpu.sync_copy(lo_ref.at[pl.ds(base, _LW)], lo_v)
    pltpu.sync_copy(hi_ref.at[pl.ds(base, _LW)], hi_v)
    pltpu.sync_copy(rad_ref.at[pl.ds(base, _LW)], rad_v)
    pltpu.sync_copy(gtb_ref.at[pl.ds(b * 4 * _M, 4 * _M)], gtb_v)
    pltpu.sync_copy(cls_ref.at[pl.ds(b * _M, _M)], clsr_v)
    pltpu.sync_copy(mw_ref.at[pl.ds((b * _NQ + q) * _LW, _LW)], mw_v)

    for gi in range(_M // 4):
        v16 = gtb_v[pl.ds(gi * 16, 16)]
        for j in range(4):
            m = gi * 4 + j
            x0 = v16[4 * j]
            y1 = v16[4 * j + 1]
            x2 = v16[4 * j + 2]
            y3 = v16[4 * j + 3]
            boxf_v[pl.ds(m * 16, 16)] = jnp.full((16,), x0, jnp.float32)
            boxf_v[pl.ds(m * 16 + _M * 16, 16)] = jnp.full((16,), y1,
                                                           jnp.float32)
            boxf_v[pl.ds(m * 16 + 2 * _M * 16, 16)] = jnp.full((16,), x2,
                                                               jnp.float32)
            boxf_v[pl.ds(m * 16 + 3 * _M * 16, 16)] = jnp.full((16,), y3,
                                                               jnp.float32)
            boxf_v[pl.ds(m * 16 + 4 * _M * 16, 16)] = jnp.full(
                (16,), (x0 + x2) * 0.5, jnp.float32)
            boxf_v[pl.ds(m * 16 + 5 * _M * 16, 16)] = jnp.full(
                (16,), (y1 + y3) * 0.5, jnp.float32)
    for gi in range(_M // 16):
        c16 = clsr_v[pl.ds(gi * 16, 16)]
        for j in range(16):
            m = gi * 16 + j
            clsb_v[pl.ds(m * 16, 16)] = jnp.full((16,), c16[j], jnp.int32)

    big = jnp.full((16,), _BIG, jnp.float32)

    def chunk(i, carry):
        s16 = pl.ds(i * 16, 16)
        xv = x_v[s16]
        yv = y_v[s16]
        lov = lo_v[s16]
        hiv = hi_v[s16]
        radv = rad_v[s16]
        w16 = mw_v[s16]
        m_a = w16[0]
        m_b = w16[1]

        zero = jnp.zeros((16,), jnp.float32)
        state = (jnp.full((16,), 2e8, jnp.float32), zero, zero, zero, zero,
                 jnp.zeros((16,), jnp.int32))

        def one_box(mb, st):
            best, sl, stt, sr, sb, scl = st
            x0 = boxf_v[pl.ds(mb, 16)]
            y1 = boxf_v[pl.ds(mb + _M * 16, 16)]
            x2 = boxf_v[pl.ds(mb + 2 * _M * 16, 16)]
            y3 = boxf_v[pl.ds(mb + 3 * _M * 16, 16)]
            cx = boxf_v[pl.ds(mb + 4 * _M * 16, 16)]
            cy = boxf_v[pl.ds(mb + 5 * _M * 16, 16)]
            cl = clsb_v[pl.ds(mb, 16)]
            l = xv - x0
            t = yv - y1
            r = x2 - xv
            bb = y3 - yv
            area = (l + r) * (t + bb)
            mn = jnp.minimum(jnp.minimum(l, t), jnp.minimum(r, bb))
            mx = jnp.maximum(jnp.maximum(l, t), jnp.maximum(r, bb))
            dm = jnp.maximum(jnp.abs(xv - cx), jnp.abs(yv - cy))
            mask = (mn > 0.0) & (mx > lov) & (mx <= hiv) & (dm < radv)
            am = jnp.where(mask, area, big)
            take = am < best
            best = jnp.where(take, am, best)
            sl = jnp.where(take, l, sl)
            stt = jnp.where(take, t, stt)
            sr = jnp.where(take, r, sr)
            sb = jnp.where(take, bb, sb)
            scl = jnp.where(take, cl, scl)
            return best, sl, stt, sr, sb, scl

        def popcount(m0):
            x = m0 - (lax.shift_right_logical(m0, 1) & 0x55555555)
            x = (x & 0x33333333) + (lax.shift_right_logical(x, 2) & 0x33333333)
            x = (x + lax.shift_right_logical(x, 4)) & 0x0F0F0F0F
            return lax.shift_right_logical(x * 0x01010101, 24)

        def bit_loop(moffset16):
            def body(k, carry):
                m0 = carry[0]
                low = m0 & (-m0)
                fb = lax.bitcast_convert_type(
                    lax.convert_element_type(low, jnp.float32), jnp.int32)
                bi = ((fb >> 23) & 255) - 127
                st2 = one_box(bi * 16 + moffset16, carry[1:])
                return (m0 ^ low,) + st2
            return body

        res = lax.fori_loop(0, popcount(m_a), bit_loop(0), (m_a,) + state)
        res = lax.fori_loop(0, popcount(m_b), bit_loop(_M // 2 * 16),
                            (m_b,) + res[1:])
        best, sl, stt, sr, sb, scl = res[1:]
        anyp = best < big
        lrmin = jnp.minimum(sl, sr)
        lrmax = jnp.maximum(sl, sr)
        tbmin = jnp.minimum(stt, sb)
        tbmax = jnp.maximum(stt, sb)
        ratio = (lrmin * tbmin) / (lrmax * tbmax + 1e-10)
        s = jnp.maximum(jnp.maximum(ratio, 0.0), 1e-30)
        bits = lax.bitcast_convert_type(s, jnp.int32)
        yv0 = lax.bitcast_convert_type(jnp.int32(0x5F3759DF) - (bits >> 1),
                                       jnp.float32)
        for _ in range(3):
            yv0 = yv0 * (1.5 - 0.5 * s * yv0 * yv0)
        sq = s * yv0
        neg1 = jnp.full((16,), -1.0, jnp.float32)
        ocnt_v[s16] = jnp.where(anyp, sq, neg1)
        ocls_v[s16] = jnp.where(anyp, scl, jnp.zeros((16,), jnp.int32))
        orl_v[s16] = jnp.where(anyp, sl, neg1)
        ort_v[s16] = jnp.where(anyp, stt, neg1)
        orr_v[s16] = jnp.where(anyp, sr, neg1)
        orb_v[s16] = jnp.where(anyp, sb, neg1)
        return carry

    lax.fori_loop(0, _NCHQ, chunk, 0)

    obase = b * _NLOC + q * _QL
    rbase = b * 4 * _NLOC + q * _QL
    qtail = _NLOC - 3 * _QL

    def _emit(n):
        pltpu.sync_copy(ocls_v.at[pl.ds(0, n)], cls_out.at[pl.ds(obase, n)])
        pltpu.sync_copy(ocnt_v.at[pl.ds(0, n)], cnt_out.at[pl.ds(obase, n)])
        pltpu.sync_copy(orl_v.at[pl.ds(0, n)], reg_out.at[pl.ds(rbase, n)])
        pltpu.sync_copy(ort_v.at[pl.ds(0, n)],
                        reg_out.at[pl.ds(rbase + _NLOC, n)])
        pltpu.sync_copy(orr_v.at[pl.ds(0, n)],
                        reg_out.at[pl.ds(rbase + 2 * _NLOC, n)])
        pltpu.sync_copy(orb_v.at[pl.ds(0, n)],
                        reg_out.at[pl.ds(rbase + 3 * _NLOC, n)])

    @pl.when(q != _NQ - 1)
    def _():
        _emit(_QL)

    @pl.when(q == _NQ - 1)
    def _():
        _emit(qtail)


@functools.partial(
    pl.kernel,
    out_type=(
        jax.ShapeDtypeStruct((_B * _NLOC,), jnp.int32),
        jax.ShapeDtypeStruct((_B * _NLOC,), jnp.float32),
        jax.ShapeDtypeStruct((_B * 4 * _NLOC,), jnp.float32),
    ),
    mesh=plsc.VectorSubcoreMesh(core_axis_name="c", subcore_axis_name="s",
                                num_cores=2, num_subcores=16),
    scratch_types=(
        [pltpu.VMEM((_LW,), jnp.float32) for _ in range(5)]
        + [
            pltpu.VMEM((4 * _M,), jnp.float32),
            pltpu.VMEM((_M,), jnp.int32),
            pltpu.VMEM((6 * _M * 16,), jnp.float32),
            pltpu.VMEM((_M * 16,), jnp.int32),
            pltpu.VMEM((_LW,), jnp.int32),
            pltpu.VMEM((_LW,), jnp.int32),
        ]
        + [pltpu.VMEM((_LW,), jnp.float32) for _ in range(5)]
    ),
)
def _gen_targets_sc(xs, ys, lo, hi, rad, gtb, cls_i, mw,
                    cls_out, cnt_out, reg_out, *scratch):
    _sc_body(xs, ys, lo, hi, rad, gtb, cls_i, mw,
             cls_out, cnt_out, reg_out, *scratch)


def kernel(cls_logits_0, cls_logits_1, cls_logits_2, cls_logits_3,
           cls_logits_4, cnt_logits_0, cnt_logits_1, cnt_logits_2,
           cnt_logits_3, cnt_logits_4, reg_preds_0, reg_preds_1, reg_preds_2,
           reg_preds_3, reg_preds_4, gt_boxes, classes):
    x0 = gt_boxes[..., 0]
    y1 = gt_boxes[..., 1]
    x2 = gt_boxes[..., 2]
    y3 = gt_boxes[..., 3]
    cx = (x0 + x2) / 2
    cy = (y1 + y3) / 2
    hw = (x2 - x0) * 0.5
    hh = (y3 - y1) * 0.5
    xcc = jnp.asarray(_XCC)[None, :, None]
    ycc = jnp.asarray(_YCC)[None, :, None]
    hsx = jnp.asarray(_HSXC)[None, :, None]
    hsy = jnp.asarray(_HSYC)[None, :, None]
    radc = jnp.asarray(_RADC)[None, :, None]
    loc = jnp.asarray(_LOC)[None, :, None]
    hic = jnp.asarray(_HIC)[None, :, None]
    cxb = cx[:, None, :]
    cyb = cy[:, None, :]
    hwb = hw[:, None, :]
    hhb = hh[:, None, :]
    act = ((jnp.abs(cxb - xcc) < jnp.minimum(radc, hwb) + hsx)
           & (jnp.abs(cyb - ycc) < jnp.minimum(radc, hhb) + hsy))
    mxwh = jnp.maximum(hwb, hhb)
    act = act & (mxwh > loc - radc - 1.0) & (mxwh <= hic + 1.0)
    pow32 = jnp.asarray(_POW32)
    m_a = jnp.sum(jnp.where(act[..., :32], pow32, jnp.uint32(0)), axis=-1,
                  dtype=jnp.uint32)
    m_b = jnp.sum(jnp.where(act[..., 32:], pow32, jnp.uint32(0)), axis=-1,
                  dtype=jnp.uint32)
    words = lax.bitcast_convert_type(jnp.stack([m_a, m_b], axis=-1),
                                     jnp.int32)
    mw = jnp.concatenate(
        [words, jnp.zeros((_B, _NCHT, 14), jnp.int32)], axis=-1
    ).reshape(_B * _NQ * _LW)

    cls_flat, cnt_flat, reg_flat = _gen_targets_sc(
        jnp.asarray(_XS), jnp.asarray(_YS), jnp.asarray(_LO),
        jnp.asarray(_HI), jnp.asarray(_RAD),
        gt_boxes.reshape(_B * 4 * _M), classes.reshape(_B * _M), mw)

    cls_t = cls_flat.reshape(_B, _NLOC, 1)
    cnt_t = cnt_flat.reshape(_B, _NLOC, 1)
    reg_t = jnp.transpose(reg_flat.reshape(_B, 4, _NLOC), (0, 2, 1))
    return cls_t, cnt_t, reg_t


# speedup vs baseline: 1.0894x; 1.0317x over previous
import functools

import numpy as np
import jax
import jax.numpy as jnp
from jax import lax
from jax.experimental import pallas as pl
from jax.experimental.pallas import tpu as pltpu
from jax.experimental.pallas import tpu_sc as plsc

_STRIDES = [8, 16, 32, 64, 128]
_LIMITS = [[-1, 64], [64, 128], [128, 256], [256, 512], [512, 999999]]
_LEVEL_HW = [(64, 64), (32, 32), (16, 16), (8, 8), (4, 4)]
_B, _M = 8, 64
_NLOC = sum(h * w for h, w in _LEVEL_HW)
_NLOCP = 5504
_NQ = 4
_QL = _NLOCP // _NQ
_NCHQ = 88
_LW = _NCHQ * 16
_NCHT = _NQ * _NCHQ
_BIG = np.float32(99999999.0)


def _pad_quarters(a, padval):
    a = a.reshape(_NQ, _QL)
    pad = np.full((_NQ, _LW - _QL), padval, np.float32)
    return np.concatenate([a, pad], axis=1).reshape(-1)


def _build_loc_tables():
    xs, ys, lo, hi, rad = [], [], [], [], []
    for (h, w), s, (llo, lhi) in zip(_LEVEL_HW, _STRIDES, _LIMITS):
        ix = np.arange(w, dtype=np.float32) * s + s // 2
        iy = np.arange(h, dtype=np.float32) * s + s // 2
        xs.append(np.tile(ix, h))
        ys.append(np.repeat(iy, w))
        lo.append(np.full(h * w, llo, np.float32))
        hi.append(np.full(h * w, lhi, np.float32))
        rad.append(np.full(h * w, s * 1.5, np.float32))
    pad = _NLOCP - _NLOC
    out = []
    for arrs, padval in zip((xs, ys, lo, hi, rad), (0.0, 0.0, 1e9, -1e9, 0.0)):
        a = np.concatenate(arrs)
        a = np.concatenate([a, np.full(pad, padval, np.float32)])
        out.append(_pad_quarters(a, padval))
    return out


_XS, _YS, _LO, _HI, _RAD = _build_loc_tables()


def _build_chunk_tables():
    xs2 = _XS.reshape(-1, 16)
    ys2 = _YS.reshape(-1, 16)
    xc = (xs2.min(axis=1) + xs2.max(axis=1)) * 0.5
    hsx = (xs2.max(axis=1) - xs2.min(axis=1)) * 0.5 + 0.5
    yc = (ys2.min(axis=1) + ys2.max(axis=1)) * 0.5
    hsy = (ys2.max(axis=1) - ys2.min(axis=1)) * 0.5 + 0.5
    rad = _RAD.reshape(-1, 16)[:, 0]
    lo = _LO.reshape(-1, 16)[:, 0]
    hi = _HI.reshape(-1, 16)[:, 0]
    f32 = lambda a: a.astype(np.float32)
    return tuple(map(f32, (xc, yc, hsx, hsy, rad, lo, hi)))


_XCC, _YCC, _HSXC, _HSYC, _RADC, _LOC, _HIC = _build_chunk_tables()
_POW32 = (np.uint32(1) << np.arange(32, dtype=np.uint32)).astype(np.uint32)


def _sc_body(xs_ref, ys_ref, lo_ref, hi_ref, rad_ref, gtb_ref, cls_ref,
             mw_ref,
             cls_out, cnt_out, reg_out,
             x_v, y_v, lo_v, hi_v, rad_v, gtb_v, clsr_v, boxf_v, clsb_v,
             mw_v, ocls_v, ocnt_v, orl_v, ort_v, orr_v, orb_v):
    cid = lax.axis_index("c")
    sid = lax.axis_index("s")
    wid = sid * 2 + cid
    b = wid // _NQ
    q = wid % _NQ
    base = q * _LW

    pltpu.sync_copy(xs_ref.at[pl.ds(base, _LW)], x_v)
    pltpu.sync_copy(ys_ref.at[pl.ds(base, _LW)], y_v)
    pltpu.sync_copy(lo_ref.at[pl.ds(base, _LW)], lo_v)
    pltpu.sync_copy(hi_ref.at[pl.ds(base, _LW)], hi_v)
    pltpu.sync_copy(rad_ref.at[pl.ds(base, _LW)], rad_v)
    pltpu.sync_copy(gtb_ref.at[pl.ds(b * 4 * _M, 4 * _M)], gtb_v)
    pltpu.sync_copy(cls_ref.at[pl.ds(b * _M, _M)], clsr_v)
    pltpu.sync_copy(mw_ref.at[pl.ds((b * _NQ + q) * _LW, _LW)], mw_v)

    for gi in range(_M // 4):
        v16 = gtb_v[pl.ds(gi * 16, 16)]
        for j in range(4):
            m = gi * 4 + j
            x0 = v16[4 * j]
            y1 = v16[4 * j + 1]
            x2 = v16[4 * j + 2]
            y3 = v16[4 * j + 3]
            boxf_v[pl.ds(m * 16, 16)] = jnp.full((16,), x0, jnp.float32)
            boxf_v[pl.ds(m * 16 + _M * 16, 16)] = jnp.full((16,), y1,
                                                           jnp.float32)
            boxf_v[pl.ds(m * 16 + 2 * _M * 16, 16)] = jnp.full((16,), x2,
                                                               jnp.float32)
            boxf_v[pl.ds(m * 16 + 3 * _M * 16, 16)] = jnp.full((16,), y3,
                                                               jnp.float32)
            boxf_v[pl.ds(m * 16 + 4 * _M * 16, 16)] = jnp.full(
                (16,), (x0 + x2) * 0.5, jnp.float32)
            boxf_v[pl.ds(m * 16 + 5 * _M * 16, 16)] = jnp.full(
                (16,), (y1 + y3) * 0.5, jnp.float32)
    for gi in range(_M // 16):
        c16 = clsr_v[pl.ds(gi * 16, 16)]
        for j in range(16):
            m = gi * 16 + j
            clsb_v[pl.ds(m * 16, 16)] = jnp.full((16,), c16[j], jnp.int32)

    big = jnp.full((16,), _BIG, jnp.float32)

    neg1 = jnp.full((16,), -1.0, jnp.float32)
    zero_i = jnp.zeros((16,), jnp.int32)

    def _rsqrt(s):
        bits = lax.bitcast_convert_type(s, jnp.int32)
        y0 = lax.bitcast_convert_type(jnp.int32(0x5F3759DF) - (bits >> 1),
                                      jnp.float32)
        for _ in range(2):
            y0 = y0 * (1.5 - 0.5 * s * y0 * y0)
        return y0

    def chunk(i, carry):
        s16 = pl.ds(i * 16, 16)
        w16 = mw_v[s16]
        m_a = w16[0]
        m_b = w16[1]

        @pl.when((m_a | m_b) == 0)
        def _():
            ocnt_v[s16] = neg1
            ocls_v[s16] = zero_i
            orl_v[s16] = neg1
            ort_v[s16] = neg1
            orr_v[s16] = neg1
            orb_v[s16] = neg1

        @pl.when((m_a | m_b) != 0)
        def _():
            xv = x_v[s16]
            yv = y_v[s16]
            lov = lo_v[s16]
            hiv = hi_v[s16]
            radv = rad_v[s16]

            zero = jnp.zeros((16,), jnp.float32)
            state = (jnp.full((16,), 2e8, jnp.float32), zero, zero, zero,
                     zero, zero_i)

            def one_box(mb, st):
                best, sl, stt, sr, sb, scl = st
                x0 = boxf_v[pl.ds(mb, 16)]
                y1 = boxf_v[pl.ds(mb + _M * 16, 16)]
                x2 = boxf_v[pl.ds(mb + 2 * _M * 16, 16)]
                y3 = boxf_v[pl.ds(mb + 3 * _M * 16, 16)]
                cx = boxf_v[pl.ds(mb + 4 * _M * 16, 16)]
                cy = boxf_v[pl.ds(mb + 5 * _M * 16, 16)]
                cl = clsb_v[pl.ds(mb, 16)]
                l = xv - x0
                t = yv - y1
                r = x2 - xv
                bb = y3 - yv
                area = (l + r) * (t + bb)
                mn = jnp.minimum(jnp.minimum(l, t), jnp.minimum(r, bb))
                mx = jnp.maximum(jnp.maximum(l, t), jnp.maximum(r, bb))
                dm = jnp.maximum(jnp.abs(xv - cx), jnp.abs(yv - cy))
                mask = (mn > 0.0) & (mx > lov) & (mx <= hiv) & (dm < radv)
                am = jnp.where(mask, area, big)
                take = am < best
                best = jnp.where(take, am, best)
                sl = jnp.where(take, l, sl)
                stt = jnp.where(take, t, stt)
                sr = jnp.where(take, r, sr)
                sb = jnp.where(take, bb, sb)
                scl = jnp.where(take, cl, scl)
                return best, sl, stt, sr, sb, scl

            def popcount(m0):
                x = m0 - (lax.shift_right_logical(m0, 1) & 0x55555555)
                x = ((x & 0x33333333)
                     + (lax.shift_right_logical(x, 2) & 0x33333333))
                x = (x + lax.shift_right_logical(x, 4)) & 0x0F0F0F0F
                return lax.shift_right_logical(x * 0x01010101, 24)

            def body(k, carry2):
                ma, mb_ = carry2[0], carry2[1]
                use_a = ma != 0
                mcur = jnp.where(use_a, ma, mb_)
                low = mcur & (-mcur)
                fb = lax.bitcast_convert_type(
                    lax.convert_element_type(low, jnp.float32), jnp.int32)
                bi = ((fb >> 23) & 255) - 127
                moff = jnp.where(use_a, 0, _M // 2 * 16)
                st2 = one_box(bi * 16 + moff, carry2[2:])
                ma2 = jnp.where(use_a, ma ^ low, ma)
                mb2 = jnp.where(use_a, mb_, mb_ ^ low)
                return (ma2, mb2) + st2

            total = popcount(m_a) + popcount(m_b)
            res = lax.fori_loop(0, total, body, (m_a, m_b) + state)
            best, sl, stt, sr, sb, scl = res[2:]
            anyp = best < big
            lrmin = jnp.minimum(sl, sr)
            lrmax = jnp.maximum(sl, sr)
            tbmin = jnp.minimum(stt, sb)
            tbmax = jnp.maximum(stt, sb)
            num = lrmin * tbmin
            den = lrmax * tbmax + 1e-10
            sq = num * _rsqrt(num) * _rsqrt(den)
            ocnt_v[s16] = jnp.where(anyp, sq, neg1)
            ocls_v[s16] = jnp.where(anyp, scl, zero_i)
            orl_v[s16] = jnp.where(anyp, sl, neg1)
            ort_v[s16] = jnp.where(anyp, stt, neg1)
            orr_v[s16] = jnp.where(anyp, sr, neg1)
            orb_v[s16] = jnp.where(anyp, sb, neg1)

        return carry

    lax.fori_loop(0, _NCHQ, chunk, 0)

    obase = b * _NLOC + q * _QL
    rbase = b * 4 * _NLOC + q * _QL
    qtail = _NLOC - 3 * _QL

    def _emit(n):
        pltpu.sync_copy(ocls_v.at[pl.ds(0, n)], cls_out.at[pl.ds(obase, n)])
        pltpu.sync_copy(ocnt_v.at[pl.ds(0, n)], cnt_out.at[pl.ds(obase, n)])
        pltpu.sync_copy(orl_v.at[pl.ds(0, n)], reg_out.at[pl.ds(rbase, n)])
        pltpu.sync_copy(ort_v.at[pl.ds(0, n)],
                        reg_out.at[pl.ds(rbase + _NLOC, n)])
        pltpu.sync_copy(orr_v.at[pl.ds(0, n)],
                        reg_out.at[pl.ds(rbase + 2 * _NLOC, n)])
        pltpu.sync_copy(orb_v.at[pl.ds(0, n)],
                        reg_out.at[pl.ds(rbase + 3 * _NLOC, n)])

    @pl.when(q != _NQ - 1)
    def _():
        _emit(_QL)

    @pl.when(q == _NQ - 1)
    def _():
        _emit(qtail)


@functools.partial(
    pl.kernel,
    out_type=(
        jax.ShapeDtypeStruct((_B * _NLOC,), jnp.int32),
        jax.ShapeDtypeStruct((_B * _NLOC,), jnp.float32),
        jax.ShapeDtypeStruct((_B * 4 * _NLOC,), jnp.float32),
    ),
    mesh=plsc.VectorSubcoreMesh(core_axis_name="c", subcore_axis_name="s",
                                num_cores=2, num_subcores=16),
    scratch_types=(
        [pltpu.VMEM((_LW,), jnp.float32) for _ in range(5)]
        + [
            pltpu.VMEM((4 * _M,), jnp.float32),
            pltpu.VMEM((_M,), jnp.int32),
            pltpu.VMEM((6 * _M * 16,), jnp.float32),
            pltpu.VMEM((_M * 16,), jnp.int32),
            pltpu.VMEM((_LW,), jnp.int32),
            pltpu.VMEM((_LW,), jnp.int32),
        ]
        + [pltpu.VMEM((_LW,), jnp.float32) for _ in range(5)]
    ),
)
def _gen_targets_sc(xs, ys, lo, hi, rad, gtb, cls_i, mw,
                    cls_out, cnt_out, reg_out, *scratch):
    _sc_body(xs, ys, lo, hi, rad, gtb, cls_i, mw,
             cls_out, cnt_out, reg_out, *scratch)


def kernel(cls_logits_0, cls_logits_1, cls_logits_2, cls_logits_3,
           cls_logits_4, cnt_logits_0, cnt_logits_1, cnt_logits_2,
           cnt_logits_3, cnt_logits_4, reg_preds_0, reg_preds_1, reg_preds_2,
           reg_preds_3, reg_preds_4, gt_boxes, classes):
    x0 = gt_boxes[..., 0]
    y1 = gt_boxes[..., 1]
    x2 = gt_boxes[..., 2]
    y3 = gt_boxes[..., 3]
    cx = (x0 + x2) / 2
    cy = (y1 + y3) / 2
    hw = (x2 - x0) * 0.5
    hh = (y3 - y1) * 0.5
    xcc = jnp.asarray(_XCC)[None, :, None]
    ycc = jnp.asarray(_YCC)[None, :, None]
    hsx = jnp.asarray(_HSXC)[None, :, None]
    hsy = jnp.asarray(_HSYC)[None, :, None]
    radc = jnp.asarray(_RADC)[None, :, None]
    loc = jnp.asarray(_LOC)[None, :, None]
    hic = jnp.asarray(_HIC)[None, :, None]
    cxb = cx[:, None, :]
    cyb = cy[:, None, :]
    hwb = hw[:, None, :]
    hhb = hh[:, None, :]
    act = ((jnp.abs(cxb - xcc) < jnp.minimum(radc, hwb) + hsx)
           & (jnp.abs(cyb - ycc) < jnp.minimum(radc, hhb) + hsy))
    mxwh = jnp.maximum(hwb, hhb)
    act = act & (mxwh > loc - radc - 1.0) & (mxwh <= hic + 1.0)
    pow32 = jnp.asarray(_POW32)
    m_a = jnp.sum(jnp.where(act[..., :32], pow32, jnp.uint32(0)), axis=-1,
                  dtype=jnp.uint32)
    m_b = jnp.sum(jnp.where(act[..., 32:], pow32, jnp.uint32(0)), axis=-1,
                  dtype=jnp.uint32)
    words = lax.bitcast_convert_type(jnp.stack([m_a, m_b], axis=-1),
                                     jnp.int32)
    mw = jnp.concatenate(
        [words, jnp.zeros((_B, _NCHT, 14), jnp.int32)], axis=-1
    ).reshape(_B * _NQ * _LW)

    cls_flat, cnt_flat, reg_flat = _gen_targets_sc(
        jnp.asarray(_XS), jnp.asarray(_YS), jnp.asarray(_LO),
        jnp.asarray(_HI), jnp.asarray(_RAD),
        gt_boxes.reshape(_B * 4 * _M), classes.reshape(_B * _M), mw)

    cls_t = cls_flat.reshape(_B, _NLOC, 1)
    cnt_t = cnt_flat.reshape(_B, _NLOC, 1)
    reg_t = jnp.transpose(reg_flat.reshape(_B, 4, _NLOC), (0, 2, 1))
    return cls_t, cnt_t, reg_t


# speedup vs baseline: 1.2155x; 1.1158x over previous
import functools

import numpy as np
import jax
import jax.numpy as jnp
from jax import lax
from jax.experimental import pallas as pl
from jax.experimental.pallas import tpu as pltpu
from jax.experimental.pallas import tpu_sc as plsc

_STRIDES = [8, 16, 32, 64, 128]
_LIMITS = [[-1, 64], [64, 128], [128, 256], [256, 512], [512, 999999]]
_LEVEL_HW = [(64, 64), (32, 32), (16, 16), (8, 8), (4, 4)]
_B, _M = 8, 64
_NLOC = sum(h * w for h, w in _LEVEL_HW)
_NLOCP = 5504
_NQ = 4
_QL = _NLOCP // _NQ
_NCHQ = 88
_LW = _NCHQ * 16
_NCHT = _NQ * _NCHQ
_BIG = np.float32(99999999.0)


def _pad_quarters(a, padval):
    a = a.reshape(_NQ, _QL)
    pad = np.full((_NQ, _LW - _QL), padval, np.float32)
    return np.concatenate([a, pad], axis=1).reshape(-1)


def _build_loc_tables():
    xs, ys, lo, hi, rad = [], [], [], [], []
    for (h, w), s, (llo, lhi) in zip(_LEVEL_HW, _STRIDES, _LIMITS):
        ix = np.arange(w, dtype=np.float32) * s + s // 2
        iy = np.arange(h, dtype=np.float32) * s + s // 2
        xs.append(np.tile(ix, h))
        ys.append(np.repeat(iy, w))
        lo.append(np.full(h * w, llo, np.float32))
        hi.append(np.full(h * w, lhi, np.float32))
        rad.append(np.full(h * w, s * 1.5, np.float32))
    pad = _NLOCP - _NLOC
    out = []
    for arrs, padval in zip((xs, ys, lo, hi, rad), (0.0, 0.0, 1e9, -1e9, 0.0)):
        a = np.concatenate(arrs)
        a = np.concatenate([a, np.full(pad, padval, np.float32)])
        out.append(_pad_quarters(a, padval))
    return out


_XS, _YS, _LO, _HI, _RAD = _build_loc_tables()

_TAB = np.stack([t.reshape(_NQ, _LW) for t in (_XS, _YS, _LO, _HI, _RAD)],
                axis=1).reshape(-1)


def _build_chunk_tables():
    xs2 = _XS.reshape(-1, 16)
    ys2 = _YS.reshape(-1, 16)
    xc = (xs2.min(axis=1) + xs2.max(axis=1)) * 0.5
    hsx = (xs2.max(axis=1) - xs2.min(axis=1)) * 0.5 + 0.5
    yc = (ys2.min(axis=1) + ys2.max(axis=1)) * 0.5
    hsy = (ys2.max(axis=1) - ys2.min(axis=1)) * 0.5 + 0.5
    rad = _RAD.reshape(-1, 16)[:, 0]
    lo = _LO.reshape(-1, 16)[:, 0]
    hi = _HI.reshape(-1, 16)[:, 0]
    f32 = lambda a: a.astype(np.float32)
    return tuple(map(f32, (xc, yc, hsx, hsy, rad, lo, hi)))


_XCC, _YCC, _HSXC, _HSYC, _RADC, _LOC, _HIC = _build_chunk_tables()
_POW32 = (np.uint32(1) << np.arange(32, dtype=np.uint32)).astype(np.uint32)


def _sc_body(tab_ref, gtb_ref, cls_ref, mw_ref,
             cls_out, cnt_out, reg_out,
             tab_v, gtb_v, clsr_v, boxf_v, clsb_v,
             mw_v, ocls_v, ocnt_v, orl_v, ort_v, orr_v, orb_v):
    cid = lax.axis_index("c")
    sid = lax.axis_index("s")
    wid = sid * 2 + cid
    b = wid // _NQ
    q = wid % _NQ

    pltpu.sync_copy(tab_ref.at[pl.ds(q * 5 * _LW, 5 * _LW)], tab_v)
    pltpu.sync_copy(gtb_ref.at[pl.ds(b * 4 * _M, 4 * _M)], gtb_v)
    pltpu.sync_copy(cls_ref.at[pl.ds(b * _M, _M)], clsr_v)
    pltpu.sync_copy(mw_ref.at[pl.ds((b * _NQ + q) * _LW, _LW)], mw_v)

    def build_boxf(gi, carry):
        v16 = gtb_v[pl.ds(gi * 16, 16)]
        mb0 = gi * 64
        for j in range(4):
            mb = mb0 + j * 16
            x0 = v16[4 * j]
            y1 = v16[4 * j + 1]
            x2 = v16[4 * j + 2]
            y3 = v16[4 * j + 3]
            boxf_v[pl.ds(mb, 16)] = jnp.full((16,), x0, jnp.float32)
            boxf_v[pl.ds(mb + _M * 16, 16)] = jnp.full((16,), y1,
                                                       jnp.float32)
            boxf_v[pl.ds(mb + 2 * _M * 16, 16)] = jnp.full((16,), x2,
                                                           jnp.float32)
            boxf_v[pl.ds(mb + 3 * _M * 16, 16)] = jnp.full((16,), y3,
                                                           jnp.float32)
            boxf_v[pl.ds(mb + 4 * _M * 16, 16)] = jnp.full(
                (16,), (x0 + x2) * 0.5, jnp.float32)
            boxf_v[pl.ds(mb + 5 * _M * 16, 16)] = jnp.full(
                (16,), (y1 + y3) * 0.5, jnp.float32)
        return carry

    def build_clsb(gi, carry):
        c16 = clsr_v[pl.ds(gi * 16, 16)]
        mb0 = gi * 256
        for j in range(16):
            clsb_v[pl.ds(mb0 + j * 16, 16)] = jnp.full((16,), c16[j],
                                                       jnp.int32)
        return carry

    lax.fori_loop(0, _M // 4, build_boxf, 0)
    lax.fori_loop(0, _M // 16, build_clsb, 0)

    big = jnp.full((16,), _BIG, jnp.float32)

    neg1 = jnp.full((16,), -1.0, jnp.float32)
    zero_i = jnp.zeros((16,), jnp.int32)

    def _rsqrt(s):
        bits = lax.bitcast_convert_type(s, jnp.int32)
        y0 = lax.bitcast_convert_type(jnp.int32(0x5F3759DF) - (bits >> 1),
                                      jnp.float32)
        for _ in range(2):
            y0 = y0 * (1.5 - 0.5 * s * y0 * y0)
        return y0

    def chunk(i, carry):
        s16 = pl.ds(i * 16, 16)
        w16 = mw_v[s16]
        m_a = w16[0]
        m_b = w16[1]

        @pl.when((m_a | m_b) == 0)
        def _():
            ocnt_v[s16] = neg1
            ocls_v[s16] = zero_i
            orl_v[s16] = neg1
            ort_v[s16] = neg1
            orr_v[s16] = neg1
            orb_v[s16] = neg1

        @pl.when((m_a | m_b) != 0)
        def _():
            xv = tab_v[pl.ds(i * 16, 16)]
            yv = tab_v[pl.ds(_LW + i * 16, 16)]
            lov = tab_v[pl.ds(2 * _LW + i * 16, 16)]
            hiv = tab_v[pl.ds(3 * _LW + i * 16, 16)]
            radv = tab_v[pl.ds(4 * _LW + i * 16, 16)]

            zero = jnp.zeros((16,), jnp.float32)
            state = (jnp.full((16,), 2e8, jnp.float32), zero, zero, zero,
                     zero, zero_i)

            def one_box(mb, st):
                best, sl, stt, sr, sb, scl = st
                x0 = boxf_v[pl.ds(mb, 16)]
                y1 = boxf_v[pl.ds(mb + _M * 16, 16)]
                x2 = boxf_v[pl.ds(mb + 2 * _M * 16, 16)]
                y3 = boxf_v[pl.ds(mb + 3 * _M * 16, 16)]
                cx = boxf_v[pl.ds(mb + 4 * _M * 16, 16)]
                cy = boxf_v[pl.ds(mb + 5 * _M * 16, 16)]
                cl = clsb_v[pl.ds(mb, 16)]
                l = xv - x0
                t = yv - y1
                r = x2 - xv
                bb = y3 - yv
                area = (l + r) * (t + bb)
                mn = jnp.minimum(jnp.minimum(l, t), jnp.minimum(r, bb))
                mx = jnp.maximum(jnp.maximum(l, t), jnp.maximum(r, bb))
                dm = jnp.maximum(jnp.abs(xv - cx), jnp.abs(yv - cy))
                mask = (mn > 0.0) & (mx > lov) & (mx <= hiv) & (dm < radv)
                am = jnp.where(mask, area, big)
                take = am < best
                best = jnp.where(take, am, best)
                sl = jnp.where(take, l, sl)
                stt = jnp.where(take, t, stt)
                sr = jnp.where(take, r, sr)
                sb = jnp.where(take, bb, sb)
                scl = jnp.where(take, cl, scl)
                return best, sl, stt, sr, sb, scl

            def popcount(m0):
                x = m0 - (lax.shift_right_logical(m0, 1) & 0x55555555)
                x = ((x & 0x33333333)
                     + (lax.shift_right_logical(x, 2) & 0x33333333))
                x = (x + lax.shift_right_logical(x, 4)) & 0x0F0F0F0F
                return lax.shift_right_logical(x * 0x01010101, 24)

            def body(k, carry2):
                ma, mb_ = carry2[0], carry2[1]
                use_a = ma != 0
                mcur = jnp.where(use_a, ma, mb_)
                low = mcur & (-mcur)
                fb = lax.bitcast_convert_type(
                    lax.convert_element_type(low, jnp.float32), jnp.int32)
                bi = ((fb >> 23) & 255) - 127
                moff = jnp.where(use_a, 0, _M // 2 * 16)
                st2 = one_box(bi * 16 + moff, carry2[2:])
                ma2 = jnp.where(use_a, ma ^ low, ma)
                mb2 = jnp.where(use_a, mb_, mb_ ^ low)
                return (ma2, mb2) + st2

            total = popcount(m_a) + popcount(m_b)
            res = lax.fori_loop(0, total, body, (m_a, m_b) + state)
            best, sl, stt, sr, sb, scl = res[2:]
            anyp = best < big
            lrmin = jnp.minimum(sl, sr)
            lrmax = jnp.maximum(sl, sr)
            tbmin = jnp.minimum(stt, sb)
            tbmax = jnp.maximum(stt, sb)
            num = lrmin * tbmin
            den = lrmax * tbmax + 1e-10
            sq = num * _rsqrt(num) * _rsqrt(den)
            ocnt_v[s16] = jnp.where(anyp, sq, neg1)
            ocls_v[s16] = jnp.where(anyp, scl, zero_i)
            orl_v[s16] = jnp.where(anyp, sl, neg1)
            ort_v[s16] = jnp.where(anyp, stt, neg1)
            orr_v[s16] = jnp.where(anyp, sr, neg1)
            orb_v[s16] = jnp.where(anyp, sb, neg1)

        return carry

    lax.fori_loop(0, _NCHQ, chunk, 0)

    obase = b * _NLOC + q * _QL
    rbase = b * 4 * _NLOC + q * _QL
    qtail = _NLOC - 3 * _QL

    def _emit(n):
        pltpu.sync_copy(ocls_v.at[pl.ds(0, n)], cls_out.at[pl.ds(obase, n)])
        pltpu.sync_copy(ocnt_v.at[pl.ds(0, n)], cnt_out.at[pl.ds(obase, n)])
        pltpu.sync_copy(orl_v.at[pl.ds(0, n)], reg_out.at[pl.ds(rbase, n)])
        pltpu.sync_copy(ort_v.at[pl.ds(0, n)],
                        reg_out.at[pl.ds(rbase + _NLOC, n)])
        pltpu.sync_copy(orr_v.at[pl.ds(0, n)],
                        reg_out.at[pl.ds(rbase + 2 * _NLOC, n)])
        pltpu.sync_copy(orb_v.at[pl.ds(0, n)],
                        reg_out.at[pl.ds(rbase + 3 * _NLOC, n)])

    @pl.when(q != _NQ - 1)
    def _():
        _emit(_QL)

    @pl.when(q == _NQ - 1)
    def _():
        _emit(qtail)


@functools.partial(
    pl.kernel,
    out_type=(
        jax.ShapeDtypeStruct((_B * _NLOC,), jnp.int32),
        jax.ShapeDtypeStruct((_B * _NLOC,), jnp.float32),
        jax.ShapeDtypeStruct((_B * 4 * _NLOC,), jnp.float32),
    ),
    mesh=plsc.VectorSubcoreMesh(core_axis_name="c", subcore_axis_name="s",
                                num_cores=2, num_subcores=16),
    scratch_types=(
        [
            pltpu.VMEM((5 * _LW,), jnp.float32),
            pltpu.VMEM((4 * _M,), jnp.float32),
            pltpu.VMEM((_M,), jnp.int32),
            pltpu.VMEM((6 * _M * 16,), jnp.float32),
            pltpu.VMEM((_M * 16,), jnp.int32),
            pltpu.VMEM((_LW,), jnp.int32),
            pltpu.VMEM((_LW,), jnp.int32),
        ]
        + [pltpu.VMEM((_LW,), jnp.float32) for _ in range(5)]
    ),
)
def _gen_targets_sc(tab, gtb, cls_i, mw,
                    cls_out, cnt_out, reg_out, *scratch):
    _sc_body(tab, gtb, cls_i, mw,
             cls_out, cnt_out, reg_out, *scratch)


def kernel(cls_logits_0, cls_logits_1, cls_logits_2, cls_logits_3,
           cls_logits_4, cnt_logits_0, cnt_logits_1, cnt_logits_2,
           cnt_logits_3, cnt_logits_4, reg_preds_0, reg_preds_1, reg_preds_2,
           reg_preds_3, reg_preds_4, gt_boxes, classes):
    x0 = gt_boxes[..., 0]
    y1 = gt_boxes[..., 1]
    x2 = gt_boxes[..., 2]
    y3 = gt_boxes[..., 3]
    cx = (x0 + x2) / 2
    cy = (y1 + y3) / 2
    hw = (x2 - x0) * 0.5
    hh = (y3 - y1) * 0.5
    xcc = jnp.asarray(_XCC)[None, :, None]
    ycc = jnp.asarray(_YCC)[None, :, None]
    hsx = jnp.asarray(_HSXC)[None, :, None]
    hsy = jnp.asarray(_HSYC)[None, :, None]
    radc = jnp.asarray(_RADC)[None, :, None]
    loc = jnp.asarray(_LOC)[None, :, None]
    hic = jnp.asarray(_HIC)[None, :, None]
    cxb = cx[:, None, :]
    cyb = cy[:, None, :]
    hwb = hw[:, None, :]
    hhb = hh[:, None, :]
    act = ((jnp.abs(cxb - xcc) < jnp.minimum(radc, hwb) + hsx)
           & (jnp.abs(cyb - ycc) < jnp.minimum(radc, hhb) + hsy))
    mxwh = jnp.maximum(hwb, hhb)
    act = act & (mxwh > loc - radc - 1.0) & (mxwh <= hic + 1.0)
    pow32 = jnp.asarray(_POW32)
    m_a = jnp.sum(jnp.where(act[..., :32], pow32, jnp.uint32(0)), axis=-1,
                  dtype=jnp.uint32)
    m_b = jnp.sum(jnp.where(act[..., 32:], pow32, jnp.uint32(0)), axis=-1,
                  dtype=jnp.uint32)
    words = lax.bitcast_convert_type(jnp.stack([m_a, m_b], axis=-1),
                                     jnp.int32)
    mw = jnp.concatenate(
        [words, jnp.zeros((_B, _NCHT, 14), jnp.int32)], axis=-1
    ).reshape(_B * _NQ * _LW)

    cls_flat, cnt_flat, reg_flat = _gen_targets_sc(
        jnp.asarray(_TAB),
        gt_boxes.reshape(_B * 4 * _M), classes.reshape(_B * _M), mw)

    cls_t = cls_flat.reshape(_B, _NLOC, 1)
    cnt_t = cnt_flat.reshape(_B, _NLOC, 1)
    reg_t = jnp.transpose(reg_flat.reshape(_B, 4, _NLOC), (0, 2, 1))
    return cls_t, cnt_t, reg_t
